# baseline (speedup 1.0000x reference)
import functools

import jax
import jax.numpy as jnp
import numpy as np
from jax import lax
from jax.experimental import pallas as pl
from jax.experimental.pallas import tpu as pltpu
from jax.experimental.pallas import tpu_sc as plsc

_FIELD_DIMS = np.array([100000] * 26, dtype=np.int64)
_OFFSETS = np.concatenate(([0], np.cumsum(_FIELD_DIMS)[:-1])).astype(np.int32)

_NC = 2
_NS = 16
_NW = _NC * _NS
_BATCH = 4096
_NF = 26
_D = 32
_TOTAL = _BATCH * _NF
_PER_W = _TOTAL // _NW
_CHUNK = 128
_NCHUNK = _PER_W // _CHUNK

_OFFS_TILED = _OFFSETS[(np.arange(_PER_W) % _NF)].reshape(_NCHUNK, _CHUNK)


def _body(x_hbm, offs_hbm, w_hbm, out_hbm, idx_v, offs_v, rows_v, sem):
    c = lax.axis_index("c")
    s = lax.axis_index("s")
    wid = s * _NC + c

    pltpu.sync_copy(x_hbm.at[wid], idx_v)
    pltpu.sync_copy(offs_hbm, offs_v)

    def add_body(i, carry):
        j = i // (_CHUNK // 16)
        k = (i % (_CHUNK // 16)) * 16
        idx_v[j, pl.ds(k, 16)] = idx_v[j, pl.ds(k, 16)] + offs_v[j, pl.ds(k, 16)]
        return carry

    lax.fori_loop(0, _NCHUNK * (_CHUNK // 16), add_body, 0)

    copies = [
        pltpu.async_copy(
            w_hbm.at[idx_v.at[j]],
            rows_v.at[pl.ds(j * _CHUNK, _CHUNK)],
            sem,
        )
        for j in range(_NCHUNK)
    ]
    for cp in copies:
        cp.wait()

    pltpu.sync_copy(rows_v, out_hbm.at[wid])


@jax.jit
def kernel(x, W):
    mesh = plsc.VectorSubcoreMesh(
        core_axis_name="c", subcore_axis_name="s", num_cores=_NC, num_subcores=_NS
    )
    x3 = x.reshape(_NW, _NCHUNK, _CHUNK)
    offs = jnp.asarray(_OFFS_TILED)
    out = pl.kernel(
        _body,
        out_type=jax.ShapeDtypeStruct((_NW, _PER_W, _D), jnp.float32),
        mesh=mesh,
        scratch_types=[
            pltpu.VMEM((_NCHUNK, _CHUNK), jnp.int32),
            pltpu.VMEM((_NCHUNK, _CHUNK), jnp.int32),
            pltpu.VMEM((_PER_W, _D), jnp.float32),
            pltpu.SemaphoreType.DMA,
        ],
        compiler_params=pltpu.CompilerParams(use_tc_tiling_on_sc=False),
    )(x3, offs, W)
    return out.reshape(_BATCH, _NF, _D)
